# Initial kernel scaffold; baseline (speedup 1.0000x reference)
import jax
import jax.numpy as jnp
from jax.experimental import pallas as pl


def kernel(conf_pred, loc_pred, conf_true, loc_true):
    raise NotImplementedError("write your pallas kernel here")


# speedup vs baseline: 1.0358x; 1.0358x over previous
import functools

import jax
import jax.numpy as jnp
from jax.experimental import pallas as pl

N = 64
D = 8732
C = 21
DC = 2184
GD = (D + DC - 1) // DC
NEG_POS_RATIO = 3
ALPHA = 1.0
I32_MIN = -(2**31)
I32_FLIP = 0x7FFFFFFF


def _pass_a(conf_ref, lab_ref, lp_ref, lt_ref,
            ml_ref, nn_ref, npr_ref, npsum_ref, locsum_ref):
    n = pl.program_id(0)
    j = pl.program_id(1)

    x = conf_ref[0]
    lab = lab_ref[0]
    lp = lp_ref[0]
    lt = lt_ref[0]

    drow = jax.lax.broadcasted_iota(jnp.int32, (DC, 1), 0) + j * DC
    dvalid = drow < D
    pos = (lab > 0) & dvalid

    m = jnp.max(x, axis=1, keepdims=True)
    s = jnp.sum(jnp.exp(x - m), axis=1, keepdims=True)
    lse = m + jnp.log(s)
    x0 = x[:, 0:1]
    cls_iota = jax.lax.broadcasted_iota(jnp.int32, (DC, C), 1)
    xl = jnp.sum(jnp.where(cls_iota == lab, x, 0.0), axis=1, keepdims=True)
    bg = lse - x0
    nll = lse - xl

    ml_ref[0] = jnp.where(pos, -jnp.inf, bg)
    nn_ref[0] = jnp.where(pos, 0.0, nll)

    npos_blk = jnp.sum(jnp.where(pos, 1.0, 0.0))
    nllpos_blk = jnp.sum(jnp.where(pos, nll, 0.0))

    diff = lt - lp
    adiff = jnp.abs(diff)
    sl1 = jnp.where(adiff < 1.0, 0.5 * diff * diff, adiff - 0.5)
    loc_blk = jnp.sum(jnp.where(pos, sl1, 0.0))

    @pl.when(j == 0)
    def _init_row():
        npr_ref[...] = jnp.zeros((1, 1, 1), jnp.float32)

    @pl.when((n == 0) & (j == 0))
    def _init_global():
        npsum_ref[...] = jnp.zeros((1, 1), jnp.float32)
        locsum_ref[...] = jnp.zeros((1, 1), jnp.float32)

    npr_ref[...] += jnp.reshape(npos_blk, (1, 1, 1))
    npsum_ref[...] += jnp.reshape(nllpos_blk, (1, 1))
    locsum_ref[...] += jnp.reshape(loc_blk, (1, 1))


def _pass_b(ml_ref, nn_ref, npr_ref, npsum_ref, locsum_ref,
            cls_ref, loc_ref):
    ml = ml_ref[...]
    kb = jax.lax.bitcast_convert_type(ml, jnp.int32)
    key = jnp.where(kb >= 0, kb, kb ^ jnp.int32(I32_FLIP))
    npr = npr_ref[...]
    k = jnp.float32(NEG_POS_RATIO) * npr

    cnt0 = jnp.sum(jnp.where(key >= 0, 1.0, 0.0), axis=1, keepdims=True)
    p0 = jnp.where(cnt0 >= k, jnp.int32(0), jnp.int32(I32_MIN))

    def body(i, p):
        cand = p | jnp.left_shift(jnp.int32(1), jnp.int32(30) - i)
        c = jnp.sum(jnp.where(key >= cand, 1.0, 0.0), axis=1, keepdims=True)
        return jnp.where(c >= k, cand, p)

    thr = jax.lax.fori_loop(0, 31, body, p0)
    neg = key >= thr
    cls_sum = npsum_ref[0, 0] + jnp.sum(jnp.where(neg, nn_ref[...], 0.0))
    npos_total = jnp.sum(npr)
    cls_ref[...] = jnp.reshape(cls_sum / npos_total, (1, 1))
    loc_ref[...] = jnp.reshape(
        jnp.float32(ALPHA) * locsum_ref[0, 0] / npos_total, (1, 1))


@jax.jit
def kernel(conf_pred, loc_pred, conf_true, loc_true):
    lab3 = conf_true.astype(jnp.int32).reshape(N, D, 1)

    ml3, nn3, npr, npsum, locsum = pl.pallas_call(
        _pass_a,
        grid=(N, GD),
        in_specs=[
            pl.BlockSpec((1, DC, C), lambda n, j: (n, j, 0)),
            pl.BlockSpec((1, DC, 1), lambda n, j: (n, j, 0)),
            pl.BlockSpec((1, DC, 4), lambda n, j: (n, j, 0)),
            pl.BlockSpec((1, DC, 4), lambda n, j: (n, j, 0)),
        ],
        out_specs=[
            pl.BlockSpec((1, DC, 1), lambda n, j: (n, j, 0)),
            pl.BlockSpec((1, DC, 1), lambda n, j: (n, j, 0)),
            pl.BlockSpec((1, 1, 1), lambda n, j: (n, 0, 0)),
            pl.BlockSpec((1, 1), lambda n, j: (0, 0)),
            pl.BlockSpec((1, 1), lambda n, j: (0, 0)),
        ],
        out_shape=[
            jax.ShapeDtypeStruct((N, D, 1), jnp.float32),
            jax.ShapeDtypeStruct((N, D, 1), jnp.float32),
            jax.ShapeDtypeStruct((N, 1, 1), jnp.float32),
            jax.ShapeDtypeStruct((1, 1), jnp.float32),
            jax.ShapeDtypeStruct((1, 1), jnp.float32),
        ],
    )(conf_pred, lab3, loc_pred, loc_true)

    cls2, loc2 = pl.pallas_call(
        _pass_b,
        in_specs=[
            pl.BlockSpec((N, D), lambda: (0, 0)),
            pl.BlockSpec((N, D), lambda: (0, 0)),
            pl.BlockSpec((N, 1), lambda: (0, 0)),
            pl.BlockSpec((1, 1), lambda: (0, 0)),
            pl.BlockSpec((1, 1), lambda: (0, 0)),
        ],
        out_specs=[
            pl.BlockSpec((1, 1), lambda: (0, 0)),
            pl.BlockSpec((1, 1), lambda: (0, 0)),
        ],
        out_shape=[
            jax.ShapeDtypeStruct((1, 1), jnp.float32),
            jax.ShapeDtypeStruct((1, 1), jnp.float32),
        ],
    )(ml3.reshape(N, D), nn3.reshape(N, D), npr.reshape(N, 1), npsum, locsum)

    return (cls2[0, 0], loc2[0, 0])


# speedup vs baseline: 4.9414x; 4.7706x over previous
import jax
import jax.numpy as jnp
from jax.experimental import pallas as pl

N = 64
D = 8732
C = 21
DC = 2176
GD = (D + DC - 1) // DC
NEG_POS_RATIO = 3
ALPHA = 1.0
I32_MIN = -(2**31)
I32_FLIP = 0x7FFFFFFF


def _pass_a(conf_ref, lab_ref, lp_ref, lt_ref,
            ml_ref, nn_ref, npr_ref, npsum_ref, locsum_ref):
    n = pl.program_id(0)
    j = pl.program_id(1)

    x = conf_ref[0]
    lab = lab_ref[0]
    lp = lp_ref[0]
    lt = lt_ref[0]

    dcol = jax.lax.broadcasted_iota(jnp.int32, (1, DC), 1) + j * DC
    dvalid = dcol < D
    pos = (lab > 0) & dvalid

    m = jnp.max(x, axis=0, keepdims=True)
    s = jnp.sum(jnp.exp(x - m), axis=0, keepdims=True)
    lse = m + jnp.log(s)
    x0 = x[0:1, :]
    cls_iota = jax.lax.broadcasted_iota(jnp.int32, (C, DC), 0)
    xl = jnp.sum(jnp.where(cls_iota == lab, x, 0.0), axis=0, keepdims=True)
    bg = lse - x0
    nll = lse - xl

    ml_ref[0] = jnp.where(pos, -jnp.inf, bg)
    nn_ref[0] = jnp.where(pos, 0.0, nll)

    npos_blk = jnp.sum(jnp.where(pos, 1.0, 0.0))
    nllpos_blk = jnp.sum(jnp.where(pos, nll, 0.0))

    diff = lt - lp
    adiff = jnp.abs(diff)
    sl1 = jnp.where(adiff < 1.0, 0.5 * diff * diff, adiff - 0.5)
    loc_blk = jnp.sum(jnp.where(pos, sl1, 0.0))

    @pl.when(j == 0)
    def _init_row():
        npr_ref[...] = jnp.zeros((1, 1, 1), jnp.float32)

    @pl.when((n == 0) & (j == 0))
    def _init_global():
        npsum_ref[...] = jnp.zeros((1, 1), jnp.float32)
        locsum_ref[...] = jnp.zeros((1, 1), jnp.float32)

    npr_ref[...] += jnp.reshape(npos_blk, (1, 1, 1))
    npsum_ref[...] += jnp.reshape(nllpos_blk, (1, 1))
    locsum_ref[...] += jnp.reshape(loc_blk, (1, 1))


def _pass_b(ml_ref, nn_ref, npr_ref, npsum_ref, locsum_ref,
            cls_ref, loc_ref):
    ml = ml_ref[...]
    kb = jax.lax.bitcast_convert_type(ml, jnp.int32)
    key = jnp.where(kb >= 0, kb, kb ^ jnp.int32(I32_FLIP))
    npr = npr_ref[...]
    k = jnp.float32(NEG_POS_RATIO) * npr

    cnt0 = jnp.sum(jnp.where(key >= 0, 1.0, 0.0), axis=1, keepdims=True)
    p0 = jnp.where(cnt0 >= k, jnp.int32(0), jnp.int32(I32_MIN))

    def body(i, p):
        cand = p | jnp.left_shift(jnp.int32(1), jnp.int32(30) - i)
        c = jnp.sum(jnp.where(key >= cand, 1.0, 0.0), axis=1, keepdims=True)
        return jnp.where(c >= k, cand, p)

    thr = jax.lax.fori_loop(0, 31, body, p0)
    neg = key >= thr
    cls_sum = npsum_ref[0, 0] + jnp.sum(jnp.where(neg, nn_ref[...], 0.0))
    npos_total = jnp.sum(npr)
    cls_ref[...] = jnp.reshape(cls_sum / npos_total, (1, 1))
    loc_ref[...] = jnp.reshape(
        jnp.float32(ALPHA) * locsum_ref[0, 0] / npos_total, (1, 1))


@jax.jit
def kernel(conf_pred, loc_pred, conf_true, loc_true):
    conf_t = jnp.transpose(conf_pred, (0, 2, 1))
    lp_t = jnp.transpose(loc_pred, (0, 2, 1))
    lt_t = jnp.transpose(loc_true, (0, 2, 1))
    lab3 = conf_true.astype(jnp.int32).reshape(N, 1, D)

    ml3, nn3, npr, npsum, locsum = pl.pallas_call(
        _pass_a,
        grid=(N, GD),
        in_specs=[
            pl.BlockSpec((1, C, DC), lambda n, j: (n, 0, j)),
            pl.BlockSpec((1, 1, DC), lambda n, j: (n, 0, j)),
            pl.BlockSpec((1, 4, DC), lambda n, j: (n, 0, j)),
            pl.BlockSpec((1, 4, DC), lambda n, j: (n, 0, j)),
        ],
        out_specs=[
            pl.BlockSpec((1, 1, DC), lambda n, j: (n, 0, j)),
            pl.BlockSpec((1, 1, DC), lambda n, j: (n, 0, j)),
            pl.BlockSpec((1, 1, 1), lambda n, j: (n, 0, 0)),
            pl.BlockSpec((1, 1), lambda n, j: (0, 0)),
            pl.BlockSpec((1, 1), lambda n, j: (0, 0)),
        ],
        out_shape=[
            jax.ShapeDtypeStruct((N, 1, D), jnp.float32),
            jax.ShapeDtypeStruct((N, 1, D), jnp.float32),
            jax.ShapeDtypeStruct((N, 1, 1), jnp.float32),
            jax.ShapeDtypeStruct((1, 1), jnp.float32),
            jax.ShapeDtypeStruct((1, 1), jnp.float32),
        ],
    )(conf_t, lab3, lp_t, lt_t)

    cls2, loc2 = pl.pallas_call(
        _pass_b,
        in_specs=[
            pl.BlockSpec((N, D), lambda: (0, 0)),
            pl.BlockSpec((N, D), lambda: (0, 0)),
            pl.BlockSpec((N, 1), lambda: (0, 0)),
            pl.BlockSpec((1, 1), lambda: (0, 0)),
            pl.BlockSpec((1, 1), lambda: (0, 0)),
        ],
        out_specs=[
            pl.BlockSpec((1, 1), lambda: (0, 0)),
            pl.BlockSpec((1, 1), lambda: (0, 0)),
        ],
        out_shape=[
            jax.ShapeDtypeStruct((1, 1), jnp.float32),
            jax.ShapeDtypeStruct((1, 1), jnp.float32),
        ],
    )(ml3.reshape(N, D), nn3.reshape(N, D), npr.reshape(N, 1), npsum, locsum)

    return (cls2[0, 0], loc2[0, 0])


# speedup vs baseline: 9.2878x; 1.8796x over previous
import jax
import jax.numpy as jnp
from jax.experimental import pallas as pl

N = 64
D = 8732
C = 21
NEG_POS_RATIO = 3
ALPHA = 1.0
I32_MIN = -(2**31)
I32_FLIP = 0x7FFFFFFF


def _pass_a(conf_ref, lab_ref, lp_ref, lt_ref,
            ml_ref, nn_ref, npr_ref, npsum_ref, locsum_ref):
    n = pl.program_id(0)

    x = conf_ref[0]
    lab = lab_ref[0]
    lp = lp_ref[0]
    lt = lt_ref[0]

    pos = lab > 0

    m = jnp.max(x, axis=0, keepdims=True)
    s = jnp.sum(jnp.exp(x - m), axis=0, keepdims=True)
    lse = m + jnp.log(s)
    x0 = x[0:1, :]
    cls_iota = jax.lax.broadcasted_iota(jnp.int32, (C, D), 0)
    xl = jnp.sum(jnp.where(cls_iota == lab, x, 0.0), axis=0, keepdims=True)
    bg = lse - x0
    nll = lse - xl

    ml_ref[0] = jnp.where(pos, -jnp.inf, bg)
    nn_ref[0] = jnp.where(pos, 0.0, nll)

    npos_blk = jnp.sum(jnp.where(pos, 1.0, 0.0))
    nllpos_blk = jnp.sum(jnp.where(pos, nll, 0.0))

    diff = lt - lp
    adiff = jnp.abs(diff)
    sl1 = jnp.where(adiff < 1.0, 0.5 * diff * diff, adiff - 0.5)
    loc_blk = jnp.sum(jnp.where(pos, sl1, 0.0))

    npr_ref[...] = jnp.reshape(npos_blk, (1, 1, 1))

    @pl.when(n == 0)
    def _init_global():
        npsum_ref[...] = jnp.zeros((1, 1), jnp.float32)
        locsum_ref[...] = jnp.zeros((1, 1), jnp.float32)

    npsum_ref[...] += jnp.reshape(nllpos_blk, (1, 1))
    locsum_ref[...] += jnp.reshape(loc_blk, (1, 1))


def _pass_b(ml_ref, nn_ref, npr_ref, npsum_ref, locsum_ref,
            cls_ref, loc_ref):
    ml = ml_ref[...]
    kb = jax.lax.bitcast_convert_type(ml, jnp.int32)
    key = jnp.where(kb >= 0, kb, kb ^ jnp.int32(I32_FLIP))
    npr = npr_ref[...]
    k = jnp.float32(NEG_POS_RATIO) * npr

    cnt0 = jnp.sum(jnp.where(key >= 0, 1.0, 0.0), axis=1, keepdims=True)
    p0 = jnp.where(cnt0 >= k, jnp.int32(0), jnp.int32(I32_MIN))

    def body(i, p):
        cand = p | jnp.left_shift(jnp.int32(1), jnp.int32(30) - i)
        c = jnp.sum(jnp.where(key >= cand, 1.0, 0.0), axis=1, keepdims=True)
        return jnp.where(c >= k, cand, p)

    thr = jax.lax.fori_loop(0, 31, body, p0)
    neg = key >= thr
    cls_sum = npsum_ref[0, 0] + jnp.sum(jnp.where(neg, nn_ref[...], 0.0))
    npos_total = jnp.sum(npr)
    cls_ref[...] = jnp.reshape(cls_sum / npos_total, (1, 1))
    loc_ref[...] = jnp.reshape(
        jnp.float32(ALPHA) * locsum_ref[0, 0] / npos_total, (1, 1))


@jax.jit
def kernel(conf_pred, loc_pred, conf_true, loc_true):
    conf_t = jnp.transpose(conf_pred, (0, 2, 1))
    lp_t = jnp.transpose(loc_pred, (0, 2, 1))
    lt_t = jnp.transpose(loc_true, (0, 2, 1))
    lab3 = conf_true.astype(jnp.int32).reshape(N, 1, D)

    ml3, nn3, npr, npsum, locsum = pl.pallas_call(
        _pass_a,
        grid=(N,),
        in_specs=[
            pl.BlockSpec((1, C, D), lambda n: (n, 0, 0)),
            pl.BlockSpec((1, 1, D), lambda n: (n, 0, 0)),
            pl.BlockSpec((1, 4, D), lambda n: (n, 0, 0)),
            pl.BlockSpec((1, 4, D), lambda n: (n, 0, 0)),
        ],
        out_specs=[
            pl.BlockSpec((1, 1, D), lambda n: (n, 0, 0)),
            pl.BlockSpec((1, 1, D), lambda n: (n, 0, 0)),
            pl.BlockSpec((1, 1, 1), lambda n: (n, 0, 0)),
            pl.BlockSpec((1, 1), lambda n: (0, 0)),
            pl.BlockSpec((1, 1), lambda n: (0, 0)),
        ],
        out_shape=[
            jax.ShapeDtypeStruct((N, 1, D), jnp.float32),
            jax.ShapeDtypeStruct((N, 1, D), jnp.float32),
            jax.ShapeDtypeStruct((N, 1, 1), jnp.float32),
            jax.ShapeDtypeStruct((1, 1), jnp.float32),
            jax.ShapeDtypeStruct((1, 1), jnp.float32),
        ],
    )(conf_t, lab3, lp_t, lt_t)

    cls2, loc2 = pl.pallas_call(
        _pass_b,
        in_specs=[
            pl.BlockSpec((N, D), lambda: (0, 0)),
            pl.BlockSpec((N, D), lambda: (0, 0)),
            pl.BlockSpec((N, 1), lambda: (0, 0)),
            pl.BlockSpec((1, 1), lambda: (0, 0)),
            pl.BlockSpec((1, 1), lambda: (0, 0)),
        ],
        out_specs=[
            pl.BlockSpec((1, 1), lambda: (0, 0)),
            pl.BlockSpec((1, 1), lambda: (0, 0)),
        ],
        out_shape=[
            jax.ShapeDtypeStruct((1, 1), jnp.float32),
            jax.ShapeDtypeStruct((1, 1), jnp.float32),
        ],
    )(ml3.reshape(N, D), nn3.reshape(N, D), npr.reshape(N, 1), npsum, locsum)

    return (cls2[0, 0], loc2[0, 0])
